# baseline (speedup 1.0000x reference)
import functools

import jax
import jax.numpy as jnp
from jax.experimental import pallas as pl
from jax.experimental.pallas import tpu as pltpu
from jax.experimental.pallas import tpu_sc as plsc

_B, _T, _D = 2, 2048, 1024
_E, _TOPK, _HID = 8, 2, 2048
_N = _B * _T
_NP = _N * _TOPK
_TM = 256
_NTILES = _NP // _TM + _E
_NROWS = _NTILES * _TM
_NW = 32


def _router_kernel(x_ref, wr_ref, br_ref, noise_ref, i_ref, s_ref):
    logits = jnp.dot(x_ref[...], wr_ref[...], preferred_element_type=jnp.float32)
    logits = logits + br_ref[...] + noise_ref[...]
    m = jnp.max(logits, axis=-1, keepdims=True)
    ex = jnp.exp(logits - m)
    sm = ex / jnp.sum(ex, axis=-1, keepdims=True)
    e_iota = jax.lax.broadcasted_iota(jnp.int32, sm.shape, 1)
    m1 = jnp.max(sm, axis=-1, keepdims=True)
    i1 = jnp.min(jnp.where(sm == m1, e_iota, _E), axis=-1, keepdims=True)
    sm2 = jnp.where(e_iota == i1, -jnp.inf, sm)
    m2 = jnp.max(sm2, axis=-1, keepdims=True)
    i2 = jnp.min(jnp.where(sm2 == m2, e_iota, _E), axis=-1, keepdims=True)
    i_ref[...] = jnp.concatenate([i1, i2], axis=1)
    s_ref[...] = jnp.concatenate([m1, m2], axis=1)


_SQRT_HALF = 0.7071067811865476


def _gelu(v):
    return 0.5 * v * (1.0 + jax.lax.erf(v * _SQRT_HALF))


def _ffn_kernel(eref, x_ref, w_ref, w1_ref, b1_ref, w2_ref, b2_ref, o_ref):
    del eref
    h = jnp.dot(x_ref[...].astype(jnp.bfloat16), w1_ref[0].astype(jnp.bfloat16),
                preferred_element_type=jnp.float32)
    h = _gelu(h + b1_ref[0])
    o = jnp.dot(h.astype(jnp.bfloat16), w2_ref[0].astype(jnp.bfloat16),
                preferred_element_type=jnp.float32)
    o_ref[...] = (o + b2_ref[0]) * w_ref[...]


def _ln_kernel(c_ref, x_ref, g_ref, b_ref, o_ref):
    c = c_ref[...].astype(jnp.float32)
    y = c[:, :_D] + c[:, _D:] + x_ref[...]
    mu = jnp.mean(y, axis=-1, keepdims=True)
    yc = y - mu
    var = jnp.mean(yc * yc, axis=-1, keepdims=True)
    o_ref[...] = yc * jax.lax.rsqrt(var + 1e-5) * g_ref[...] + b_ref[...]


def _sc_gather(table, idx, nrows):
    ncols = table.shape[1]
    b_per_w = nrows // _NW
    ch = 16
    nbuf = 4
    n_chunks = b_per_w // ch
    mesh = plsc.VectorSubcoreMesh(core_axis_name="c", subcore_axis_name="s")

    @functools.partial(
        pl.kernel, mesh=mesh,
        out_type=jax.ShapeDtypeStruct((nrows, ncols), table.dtype),
        scratch_types=(
            [pltpu.VMEM((b_per_w,), jnp.int32)]
            + [pltpu.VMEM((ch, ncols), table.dtype)] * nbuf
            + [pltpu.SemaphoreType.DMA] * nbuf
        ),
    )
    def k(table_hbm, idx_hbm, out_hbm, idx_v, *bufsem):
        bufs = bufsem[:nbuf]
        sems = bufsem[nbuf:]
        wid = jax.lax.axis_index("s") * 2 + jax.lax.axis_index("c")
        base = wid * b_per_w
        pltpu.sync_copy(idx_hbm.at[pl.ds(base, b_per_w)], idx_v)
        handles = [None] * nbuf
        for c in range(min(nbuf, n_chunks)):
            handles[c] = pltpu.async_copy(
                table_hbm.at[idx_v.at[pl.ds(c * ch, ch)]], bufs[c], sems[c])
        for c in range(n_chunks):
            nxt = c + nbuf
            handles[c % nbuf].wait()
            pltpu.sync_copy(bufs[c % nbuf], out_hbm.at[pl.ds(base + c * ch, ch)])
            if nxt < n_chunks:
                handles[nxt % nbuf] = pltpu.async_copy(
                    table_hbm.at[idx_v.at[pl.ds(nxt * ch, ch)]],
                    bufs[nxt % nbuf], sems[nxt % nbuf])

    return k(table, idx)


def kernel(x, Wr, br, W1, b1, W2, b2, gamma, beta):
    xf = x.reshape(_N, _D)
    noise = jax.random.normal(jax.random.key(42), (_N, _E), jnp.float32) / 10.0

    topk_idx, topk_scores = pl.pallas_call(
        _router_kernel,
        grid=(_N // _TM,),
        in_specs=[
            pl.BlockSpec((_TM, _D), lambda t: (t, 0)),
            pl.BlockSpec((_D, _E), lambda t: (0, 0)),
            pl.BlockSpec((1, _E), lambda t: (0, 0)),
            pl.BlockSpec((_TM, _E), lambda t: (t, 0)),
        ],
        out_specs=[
            pl.BlockSpec((_TM, _TOPK), lambda t: (t, 0)),
            pl.BlockSpec((_TM, _TOPK), lambda t: (t, 0)),
        ],
        out_shape=[
            jax.ShapeDtypeStruct((_N, _TOPK), jnp.int32),
            jax.ShapeDtypeStruct((_N, _TOPK), jnp.float32),
        ],
    )(xf, Wr, br.reshape(1, _E), noise)

    flat_e = topk_idx.reshape(-1)
    oh = (flat_e[:, None] == jnp.arange(_E, dtype=jnp.int32)[None, :]).astype(jnp.int32)
    ranks = jnp.cumsum(oh, axis=0) - 1
    rank = jnp.take_along_axis(ranks, flat_e[:, None], axis=1)[:, 0]
    counts = jnp.sum(oh, axis=0)
    tiles_per_e = (counts + _TM - 1) // _TM
    tile_end = jnp.cumsum(tiles_per_e)
    aligned_offset = (tile_end - tiles_per_e) * _TM
    slot = aligned_offset[flat_e] + rank
    row_token = jnp.zeros((_NROWS,), jnp.int32).at[slot].set(
        jnp.arange(_NP, dtype=jnp.int32) // _TOPK)
    row_w = jnp.zeros((_NROWS, 1), jnp.float32).at[slot, 0].set(
        topk_scores.reshape(-1))
    expert_of_tile = jnp.minimum(
        jnp.sum(jnp.arange(_NTILES, dtype=jnp.int32)[:, None] >= tile_end[None, :],
                axis=1), _E - 1).astype(jnp.int32)

    x_sorted = _sc_gather(xf, row_token, _NROWS)

    ffn_out = pl.pallas_call(
        _ffn_kernel,
        grid_spec=pltpu.PrefetchScalarGridSpec(
            num_scalar_prefetch=1,
            grid=(_NTILES,),
            in_specs=[
                pl.BlockSpec((_TM, _D), lambda j, eref: (j, 0)),
                pl.BlockSpec((_TM, 1), lambda j, eref: (j, 0)),
                pl.BlockSpec((1, _D, _HID), lambda j, eref: (eref[j], 0, 0)),
                pl.BlockSpec((1, 1, _HID), lambda j, eref: (eref[j], 0, 0)),
                pl.BlockSpec((1, _HID, _D), lambda j, eref: (eref[j], 0, 0)),
                pl.BlockSpec((1, 1, _D), lambda j, eref: (eref[j], 0, 0)),
            ],
            out_specs=pl.BlockSpec((_TM, _D), lambda j, eref: (j, 0)),
        ),
        out_shape=jax.ShapeDtypeStruct((_NROWS, _D), jnp.float32),
        compiler_params=pltpu.CompilerParams(
            vmem_limit_bytes=100 * 1024 * 1024,
        ),
    )(expert_of_tile, x_sorted, row_w, W1, b1.reshape(_E, 1, _HID), W2,
      b2.reshape(_E, 1, _D))

    contrib = _sc_gather(ffn_out, slot, _NP).reshape(_N, _TOPK * _D)

    y = pl.pallas_call(
        _ln_kernel,
        grid=(_N // _TM,),
        in_specs=[
            pl.BlockSpec((_TM, _TOPK * _D), lambda t: (t, 0)),
            pl.BlockSpec((_TM, _D), lambda t: (t, 0)),
            pl.BlockSpec((1, _D), lambda t: (0, 0)),
            pl.BlockSpec((1, _D), lambda t: (0, 0)),
        ],
        out_specs=pl.BlockSpec((_TM, _D), lambda t: (t, 0)),
        out_shape=jax.ShapeDtypeStruct((_N, _D), jnp.float32),
    )(contrib, xf, gamma.reshape(1, _D), beta.reshape(1, _D))

    return y.reshape(_B, _T, _D)


# speedup vs baseline: 1.0183x; 1.0183x over previous
import functools

import jax
import jax.numpy as jnp
from jax.experimental import pallas as pl
from jax.experimental.pallas import tpu as pltpu
from jax.experimental.pallas import tpu_sc as plsc

_B, _T, _D = 2, 2048, 1024
_E, _TOPK, _HID = 8, 2, 2048
_N = _B * _T
_NP = _N * _TOPK
_TM = 256
_NTILES = _NP // _TM + _E
_NROWS = _NTILES * _TM
_NW = 32


def _router_kernel(x_ref, wr_ref, br_ref, noise_ref, i_ref, s_ref):
    logits = jnp.dot(x_ref[...], wr_ref[...], preferred_element_type=jnp.float32)
    logits = logits + br_ref[...] + noise_ref[...]
    m = jnp.max(logits, axis=-1, keepdims=True)
    ex = jnp.exp(logits - m)
    sm = ex / jnp.sum(ex, axis=-1, keepdims=True)
    e_iota = jax.lax.broadcasted_iota(jnp.int32, sm.shape, 1)
    m1 = jnp.max(sm, axis=-1, keepdims=True)
    i1 = jnp.min(jnp.where(sm == m1, e_iota, _E), axis=-1, keepdims=True)
    sm2 = jnp.where(e_iota == i1, -jnp.inf, sm)
    m2 = jnp.max(sm2, axis=-1, keepdims=True)
    i2 = jnp.min(jnp.where(sm2 == m2, e_iota, _E), axis=-1, keepdims=True)
    i_ref[...] = jnp.concatenate([i1, i2], axis=1)
    s_ref[...] = jnp.concatenate([m1, m2], axis=1)


_SQRT_HALF = 0.7071067811865476


def _gelu(v):
    return 0.5 * v * (1.0 + jax.lax.erf(v * _SQRT_HALF))


def _ffn_kernel(eref, x_ref, w1_ref, b1_ref, w2_ref, b2_ref, o_ref):
    del eref
    h = jnp.dot(x_ref[...].astype(jnp.bfloat16), w1_ref[0].astype(jnp.bfloat16),
                preferred_element_type=jnp.float32)
    h = _gelu(h + b1_ref[0])
    o = jnp.dot(h.astype(jnp.bfloat16), w2_ref[0].astype(jnp.bfloat16),
                preferred_element_type=jnp.float32)
    o_ref[...] = o + b2_ref[0]


def _ln_kernel(c_ref, s_ref, x_ref, g_ref, b_ref, o_ref):
    c = c_ref[...].astype(jnp.float32)
    s = s_ref[...]
    y = c[:, :_D] * s[:, 0:1] + c[:, _D:] * s[:, 1:2] + x_ref[...]
    mu = jnp.mean(y, axis=-1, keepdims=True)
    yc = y - mu
    var = jnp.mean(yc * yc, axis=-1, keepdims=True)
    o_ref[...] = yc * jax.lax.rsqrt(var + 1e-5) * g_ref[...] + b_ref[...]


def _meta_kernel(idx_ref, slot_ref, eot_ref):
    n_chunks = _N // _TM
    erow = jax.lax.broadcasted_iota(jnp.int32, (1, _E), 1)
    r_i = jax.lax.broadcasted_iota(jnp.int32, (_TM, _TM), 0)
    c_i = jax.lax.broadcasted_iota(jnp.int32, (_TM, _TM), 1)
    ltri = (r_i > c_i).astype(jnp.float32)
    base = jnp.zeros((1, _E), jnp.float32)
    ranks = []
    ohs = []
    for c in range(n_chunks):
        blk = idx_ref[c * _TM:(c + 1) * _TM, :]
        oh1 = (blk[:, 0:1] == erow).astype(jnp.float32)
        oh2 = (blk[:, 1:2] == erow).astype(jnp.float32)
        before = jnp.dot(ltri, oh1 + oh2, preferred_element_type=jnp.float32)
        r1 = before + base
        r2 = r1 + oh1
        rank1 = jnp.sum(r1 * oh1, axis=1, keepdims=True)
        rank2 = jnp.sum(r2 * oh2, axis=1, keepdims=True)
        ranks.append((rank1, rank2))
        ohs.append((oh1, oh2))
        base = base + jnp.sum(oh1 + oh2, axis=0, keepdims=True)
    tiles_per_e = jnp.ceil(base / _TM)
    l8 = (jax.lax.broadcasted_iota(jnp.int32, (_E, _E), 0)
          < jax.lax.broadcasted_iota(jnp.int32, (_E, _E), 1)).astype(jnp.float32)
    tile_start = jnp.dot(tiles_per_e, l8, preferred_element_type=jnp.float32)
    offset = tile_start * _TM
    tile_end = tile_start + tiles_per_e
    j40 = jax.lax.broadcasted_iota(jnp.int32, (_NTILES, _E), 0).astype(jnp.float32)
    eot = jnp.sum((j40 >= tile_end).astype(jnp.int32), axis=1, keepdims=True)
    eot_ref[...] = jnp.minimum(eot, _E - 1)
    for c in range(n_chunks):
        oh1, oh2 = ohs[c]
        rank1, rank2 = ranks[c]
        off1 = jnp.sum(offset * oh1, axis=1, keepdims=True)
        off2 = jnp.sum(offset * oh2, axis=1, keepdims=True)
        s1 = (off1 + rank1).astype(jnp.int32)
        s2 = (off2 + rank2).astype(jnp.int32)
        slot_ref[c * _TM:(c + 1) * _TM, :] = jnp.concatenate([s1, s2], axis=1)


def _sc_gather(table, idx, nrows):
    ncols = table.shape[1]
    b_per_w = nrows // _NW
    ch = 16
    nbuf = 4
    n_chunks = b_per_w // ch
    mesh = plsc.VectorSubcoreMesh(core_axis_name="c", subcore_axis_name="s")

    @functools.partial(
        pl.kernel, mesh=mesh,
        out_type=jax.ShapeDtypeStruct((nrows, ncols), table.dtype),
        scratch_types=(
            [pltpu.VMEM((b_per_w,), jnp.int32)]
            + [pltpu.VMEM((ch, ncols), table.dtype)] * nbuf
            + [pltpu.SemaphoreType.DMA] * nbuf
        ),
    )
    def k(table_hbm, idx_hbm, out_hbm, idx_v, *bufsem):
        bufs = bufsem[:nbuf]
        sems = bufsem[nbuf:]
        wid = jax.lax.axis_index("s") * 2 + jax.lax.axis_index("c")
        base = wid * b_per_w
        pltpu.sync_copy(idx_hbm.at[pl.ds(base, b_per_w)], idx_v)
        handles = [None] * nbuf
        for c in range(min(nbuf, n_chunks)):
            handles[c] = pltpu.async_copy(
                table_hbm.at[idx_v.at[pl.ds(c * ch, ch)]], bufs[c], sems[c])
        for c in range(n_chunks):
            nxt = c + nbuf
            handles[c % nbuf].wait()
            pltpu.sync_copy(bufs[c % nbuf], out_hbm.at[pl.ds(base + c * ch, ch)])
            if nxt < n_chunks:
                handles[nxt % nbuf] = pltpu.async_copy(
                    table_hbm.at[idx_v.at[pl.ds(nxt * ch, ch)]],
                    bufs[nxt % nbuf], sems[nxt % nbuf])

    return k(table, idx)


def kernel(x, Wr, br, W1, b1, W2, b2, gamma, beta):
    xf = x.reshape(_N, _D)
    noise = jax.random.normal(jax.random.key(42), (_N, _E), jnp.float32) / 10.0

    topk_idx, topk_scores = pl.pallas_call(
        _router_kernel,
        grid=(_N // _TM,),
        in_specs=[
            pl.BlockSpec((_TM, _D), lambda t: (t, 0)),
            pl.BlockSpec((_D, _E), lambda t: (0, 0)),
            pl.BlockSpec((1, _E), lambda t: (0, 0)),
            pl.BlockSpec((_TM, _E), lambda t: (t, 0)),
        ],
        out_specs=[
            pl.BlockSpec((_TM, _TOPK), lambda t: (t, 0)),
            pl.BlockSpec((_TM, _TOPK), lambda t: (t, 0)),
        ],
        out_shape=[
            jax.ShapeDtypeStruct((_N, _TOPK), jnp.int32),
            jax.ShapeDtypeStruct((_N, _TOPK), jnp.float32),
        ],
    )(xf, Wr, br.reshape(1, _E), noise)

    slot2, eot2 = pl.pallas_call(
        _meta_kernel,
        in_specs=[pl.BlockSpec((_N, _TOPK), lambda: (0, 0))],
        out_specs=[
            pl.BlockSpec((_N, _TOPK), lambda: (0, 0)),
            pl.BlockSpec((_NTILES, 1), lambda: (0, 0)),
        ],
        out_shape=[
            jax.ShapeDtypeStruct((_N, _TOPK), jnp.int32),
            jax.ShapeDtypeStruct((_NTILES, 1), jnp.int32),
        ],
    )(topk_idx)
    slot = slot2.reshape(-1)
    expert_of_tile = eot2.reshape(-1)
    row_token = jnp.zeros((_NROWS,), jnp.int32).at[slot].set(
        jnp.arange(_NP, dtype=jnp.int32) // _TOPK)

    x_sorted = _sc_gather(xf, row_token, _NROWS)

    ffn_out = pl.pallas_call(
        _ffn_kernel,
        grid_spec=pltpu.PrefetchScalarGridSpec(
            num_scalar_prefetch=1,
            grid=(_NTILES,),
            in_specs=[
                pl.BlockSpec((_TM, _D), lambda j, eref: (j, 0)),
                pl.BlockSpec((1, _D, _HID), lambda j, eref: (eref[j], 0, 0)),
                pl.BlockSpec((1, 1, _HID), lambda j, eref: (eref[j], 0, 0)),
                pl.BlockSpec((1, _HID, _D), lambda j, eref: (eref[j], 0, 0)),
                pl.BlockSpec((1, 1, _D), lambda j, eref: (eref[j], 0, 0)),
            ],
            out_specs=pl.BlockSpec((_TM, _D), lambda j, eref: (j, 0)),
        ),
        out_shape=jax.ShapeDtypeStruct((_NROWS, _D), jnp.float32),
        compiler_params=pltpu.CompilerParams(
            vmem_limit_bytes=100 * 1024 * 1024,
        ),
    )(expert_of_tile, x_sorted, W1, b1.reshape(_E, 1, _HID), W2,
      b2.reshape(_E, 1, _D))

    contrib = _sc_gather(ffn_out, slot, _NP).reshape(_N, _TOPK * _D)

    y = pl.pallas_call(
        _ln_kernel,
        grid=(_N // _TM,),
        in_specs=[
            pl.BlockSpec((_TM, _TOPK * _D), lambda t: (t, 0)),
            pl.BlockSpec((_TM, _TOPK), lambda t: (t, 0)),
            pl.BlockSpec((_TM, _D), lambda t: (t, 0)),
            pl.BlockSpec((1, _D), lambda t: (0, 0)),
            pl.BlockSpec((1, _D), lambda t: (0, 0)),
        ],
        out_specs=pl.BlockSpec((_TM, _D), lambda t: (t, 0)),
        out_shape=jax.ShapeDtypeStruct((_N, _D), jnp.float32),
    )(contrib, topk_scores, xf, gamma.reshape(1, _D), beta.reshape(1, _D))

    return y.reshape(_B, _T, _D)
